# baseline (speedup 1.0000x reference)
import functools
import math

import jax
import jax.numpy as jnp
from jax import lax
from jax.experimental import pallas as pl
from jax.experimental.pallas import tpu as pltpu
from jax.experimental.pallas import tpu_sc as plsc

EMB = 64
SCALE = math.sqrt(EMB)

_NC = 2
_NS = 16
_NW = _NC * _NS

_CHUNK = 640
_NBUF = 3


def _tile_body(table_hbm, x_hbm, out_hbm, idx_v, rows_v, isem, gsem, wsem,
               *, b_per_w, n_chunks):
  wid = lax.axis_index("s") * _NC + lax.axis_index("c")
  base = wid * b_per_w

  def start_idx(c, b):
    pltpu.async_copy(x_hbm.at[pl.ds(base + c * _CHUNK, _CHUNK)],
                     idx_v.at[b], isem[b])

  def wait_idx(b):
    pltpu.make_async_copy(x_hbm.at[pl.ds(0, _CHUNK)], idx_v.at[b],
                          isem[b]).wait()

  def start_gather(b):
    pltpu.async_copy(table_hbm.at[idx_v.at[b]], rows_v.at[b], gsem[b])

  def wait_gather(b):
    pltpu.make_async_copy(table_hbm.at[idx_v.at[b]], rows_v.at[b], gsem[b]).wait()

  def start_write(c, b):
    pltpu.async_copy(rows_v.at[b], out_hbm.at[pl.ds(base + c * _CHUNK, _CHUNK)],
                     wsem[b])

  def wait_write(b):
    pltpu.make_async_copy(rows_v.at[b], out_hbm.at[pl.ds(0, _CHUNK)],
                          wsem[b]).wait()

  def scale(b):
    rv = rows_v.at[b]

    def scale4(r, carry):
      for rr in range(4):
        for j in range(EMB // 16):
          sl = pl.ds(j * 16, 16)
          rv[r * 4 + rr, sl] = rv[r * 4 + rr, sl] * SCALE
      return carry

    lax.fori_loop(0, _CHUNK // 4, scale4, 0)

  def step(c, b, *, do_idx=True, do_gather=True, do_wwait=True):
    b1 = (b + 1) % _NBUF
    b2 = (b + 2) % _NBUF
    if do_idx:
      start_idx(c + 2, b2)
    if do_gather:
      wait_idx(b1)
      if do_wwait:
        wait_write(b1)
      start_gather(b1)
    wait_gather(b)
    scale(b)
    start_write(c, b)

  start_idx(0, 0)
  start_idx(1, 1)
  wait_idx(0)
  start_gather(0)
  step(0, 0, do_wwait=False)
  step(1, 1, do_wwait=False)
  step(2, 2)

  def group(p, carry):
    for b in range(_NBUF):
      step(p * 3 + b, b)
    return carry

  lax.fori_loop(1, (n_chunks - 4) // 3, group, 0)

  c0 = n_chunks - 4
  step(c0, 0)
  step(c0 + 1, 1)
  step(c0 + 2, 2, do_idx=False)
  step(c0 + 3, 0, do_idx=False, do_gather=False)

  wait_write(1)
  wait_write(2)
  wait_write(0)


@jax.jit
def _lookup(table, xf):
  b = xf.shape[0]
  assert b % _NW == 0
  b_per_w = b // _NW
  assert b_per_w % _CHUNK == 0
  n_chunks = b_per_w // _CHUNK
  assert n_chunks % 3 == 1 and n_chunks > 4

  mesh = plsc.VectorSubcoreMesh(core_axis_name="c", subcore_axis_name="s")
  return pl.kernel(
      functools.partial(_tile_body, b_per_w=b_per_w, n_chunks=n_chunks),
      out_type=jax.ShapeDtypeStruct((b, EMB), jnp.float32),
      mesh=mesh,
      scratch_types=[
          pltpu.VMEM((_NBUF, _CHUNK), jnp.int32),
          pltpu.VMEM((_NBUF, _CHUNK, EMB), jnp.float32),
          [pltpu.SemaphoreType.DMA] * _NBUF,
          [pltpu.SemaphoreType.DMA] * _NBUF,
          [pltpu.SemaphoreType.DMA] * _NBUF,
      ],
      compiler_params=pltpu.CompilerParams(use_tc_tiling_on_sc=False),
  )(table, xf)


def kernel(x, table):
  s, t = x.shape
  out = _lookup(table, x.reshape(s * t))
  return out.reshape(s, t, EMB)
